# baseline (speedup 1.0000x reference)
import jax
import jax.numpy as jnp
from jax.experimental import pallas as pl
from jax.experimental.pallas import tpu as pltpu

_CHUNKS = 16


def _gcm_kernel(nn_ref, x_ref, W_ref, nodes_hbm, adj_hbm, w_hbm,
                mx_ref, out_hbm, buf, arow, wrow, in_sems, out_sems,
                row_sems):
    Bsz, F = x_ref.shape
    rows_c = buf.shape[1]
    N = arow.shape[1]
    per = Bsz // _CHUNKS

    for c in range(_CHUNKS):
        pltpu.make_async_copy(nodes_hbm.at[pl.ds(c * rows_c, rows_c)],
                              buf.at[c], in_sems.at[c]).start()
    for b in range(Bsz):
        idx = nn_ref[b]
        pltpu.make_async_copy(adj_hbm.at[b, idx], arow.at[b],
                              row_sems.at[b]).start()
        pltpu.make_async_copy(w_hbm.at[b, idx], wrow.at[b],
                              row_sems.at[Bsz + b]).start()

    sub_ids = jax.lax.broadcasted_iota(jnp.int32, (8, 1), 0)
    for c in range(_CHUNKS):
        pltpu.make_async_copy(nodes_hbm.at[pl.ds(c * rows_c, rows_c)],
                              buf.at[c], in_sems.at[c]).wait()
        for jj in range(per):
            b = per * c + jj
            idx = nn_ref[b]
            grp = N * jj + (idx // 8) * 8
            sub = buf[c, pl.ds(grp, 8), :]
            buf[c, pl.ds(grp, 8), :] = jnp.where(sub_ids == idx % 8,
                                                 x_ref[b], sub)
        pltpu.make_async_copy(buf.at[c],
                              out_hbm.at[pl.ds(c * rows_c, rows_c)],
                              out_sems.at[c]).start()
        for jj in range(per):
            b = per * c + jj
            idx = nn_ref[b]
            pltpu.make_async_copy(adj_hbm.at[b, idx], arow.at[b],
                                  row_sems.at[b]).wait()
            pltpu.make_async_copy(w_hbm.at[b, idx], wrow.at[b],
                                  row_sems.at[Bsz + b]).wait()
        rows = arow[per * c:per * (c + 1)] * wrow[per * c:per * (c + 1)]
        vs = jnp.concatenate(
            [jnp.dot(rows[jj][None, :], buf[c, N * jj:N * (jj + 1), :],
                     preferred_element_type=jnp.float32)
             for jj in range(per)], axis=0)
        mx_ref[per * c:per * (c + 1)] = jnp.tanh(
            jnp.dot(vs, W_ref[...], preferred_element_type=jnp.float32))

    for c in range(_CHUNKS):
        pltpu.make_async_copy(buf.at[c],
                              out_hbm.at[pl.ds(c * rows_c, rows_c)],
                              out_sems.at[c]).wait()


def kernel(x, nodes, adj, weights, num_nodes, W):
    Bsz, N, F = nodes.shape
    nn = num_nodes.astype(jnp.int32)
    nodes_flat = nodes.reshape(Bsz * N, F)
    rows_c = (Bsz * N) // _CHUNKS

    grid_spec = pltpu.PrefetchScalarGridSpec(
        num_scalar_prefetch=1,
        grid=(1,),
        in_specs=[
            pl.BlockSpec((Bsz, F), lambda i, nn: (0, 0)),
            pl.BlockSpec((F, F), lambda i, nn: (0, 0)),
            pl.BlockSpec(memory_space=pltpu.MemorySpace.HBM),
            pl.BlockSpec(memory_space=pltpu.MemorySpace.HBM),
            pl.BlockSpec(memory_space=pltpu.MemorySpace.HBM),
        ],
        out_specs=[
            pl.BlockSpec((Bsz, F), lambda i, nn: (0, 0)),
            pl.BlockSpec(memory_space=pltpu.MemorySpace.HBM),
        ],
        scratch_shapes=[
            pltpu.VMEM((_CHUNKS, rows_c, F), jnp.float32),
            pltpu.VMEM((Bsz, N), jnp.float32),
            pltpu.VMEM((Bsz, N), jnp.float32),
            pltpu.SemaphoreType.DMA((_CHUNKS,)),
            pltpu.SemaphoreType.DMA((_CHUNKS,)),
            pltpu.SemaphoreType.DMA((2 * Bsz,)),
        ],
    )

    mx, nodes_new_flat = pl.pallas_call(
        _gcm_kernel,
        grid_spec=grid_spec,
        out_shape=[
            jax.ShapeDtypeStruct((Bsz, F), jnp.float32),
            jax.ShapeDtypeStruct((Bsz * N, F), jnp.float32),
        ],
    )(nn, x, W, nodes_flat, adj, weights)

    return (mx, nodes_new_flat.reshape(Bsz, N, F), adj, weights,
            num_nodes + 1)


# speedup vs baseline: 1.0115x; 1.0115x over previous
import jax
import jax.numpy as jnp
from jax.experimental import pallas as pl
from jax.experimental.pallas import tpu as pltpu

_CHUNKS = 8


def _gcm_kernel(nn_ref, x_ref, W_ref, nodes_hbm, adj_hbm, w_hbm,
                mx_ref, out_hbm, buf, arow, wrow, in_sems, out_sems,
                row_sems):
    Bsz, F = x_ref.shape
    rows_c = buf.shape[1]
    N = arow.shape[1]
    per = Bsz // _CHUNKS

    for c in range(_CHUNKS):
        pltpu.make_async_copy(nodes_hbm.at[pl.ds(c * rows_c, rows_c)],
                              buf.at[c], in_sems.at[c]).start()
    for b in range(Bsz):
        idx = nn_ref[b]
        pltpu.make_async_copy(adj_hbm.at[b, idx], arow.at[b],
                              row_sems.at[b]).start()
        pltpu.make_async_copy(w_hbm.at[b, idx], wrow.at[b],
                              row_sems.at[Bsz + b]).start()

    sub_ids = jax.lax.broadcasted_iota(jnp.int32, (8, 1), 0)
    for c in range(_CHUNKS):
        pltpu.make_async_copy(nodes_hbm.at[pl.ds(c * rows_c, rows_c)],
                              buf.at[c], in_sems.at[c]).wait()
        for jj in range(per):
            b = per * c + jj
            idx = nn_ref[b]
            grp = N * jj + (idx // 8) * 8
            sub = buf[c, pl.ds(grp, 8), :]
            buf[c, pl.ds(grp, 8), :] = jnp.where(sub_ids == idx % 8,
                                                 x_ref[b], sub)
        pltpu.make_async_copy(buf.at[c],
                              out_hbm.at[pl.ds(c * rows_c, rows_c)],
                              out_sems.at[c]).start()
        for jj in range(per):
            b = per * c + jj
            idx = nn_ref[b]
            pltpu.make_async_copy(adj_hbm.at[b, idx], arow.at[b],
                                  row_sems.at[b]).wait()
            pltpu.make_async_copy(w_hbm.at[b, idx], wrow.at[b],
                                  row_sems.at[Bsz + b]).wait()
        rows = arow[per * c:per * (c + 1)] * wrow[per * c:per * (c + 1)]
        vs = jnp.concatenate(
            [jnp.dot(rows[jj][None, :], buf[c, N * jj:N * (jj + 1), :],
                     preferred_element_type=jnp.float32)
             for jj in range(per)], axis=0)
        mx_ref[per * c:per * (c + 1)] = jnp.tanh(
            jnp.dot(vs, W_ref[...], preferred_element_type=jnp.float32))

    for c in range(_CHUNKS):
        pltpu.make_async_copy(buf.at[c],
                              out_hbm.at[pl.ds(c * rows_c, rows_c)],
                              out_sems.at[c]).wait()


def kernel(x, nodes, adj, weights, num_nodes, W):
    Bsz, N, F = nodes.shape
    nn = num_nodes.astype(jnp.int32)
    nodes_flat = nodes.reshape(Bsz * N, F)
    rows_c = (Bsz * N) // _CHUNKS

    grid_spec = pltpu.PrefetchScalarGridSpec(
        num_scalar_prefetch=1,
        grid=(1,),
        in_specs=[
            pl.BlockSpec((Bsz, F), lambda i, nn: (0, 0)),
            pl.BlockSpec((F, F), lambda i, nn: (0, 0)),
            pl.BlockSpec(memory_space=pltpu.MemorySpace.HBM),
            pl.BlockSpec(memory_space=pltpu.MemorySpace.HBM),
            pl.BlockSpec(memory_space=pltpu.MemorySpace.HBM),
        ],
        out_specs=[
            pl.BlockSpec((Bsz, F), lambda i, nn: (0, 0)),
            pl.BlockSpec(memory_space=pltpu.MemorySpace.HBM),
        ],
        scratch_shapes=[
            pltpu.VMEM((_CHUNKS, rows_c, F), jnp.float32),
            pltpu.VMEM((Bsz, N), jnp.float32),
            pltpu.VMEM((Bsz, N), jnp.float32),
            pltpu.SemaphoreType.DMA((_CHUNKS,)),
            pltpu.SemaphoreType.DMA((_CHUNKS,)),
            pltpu.SemaphoreType.DMA((2 * Bsz,)),
        ],
    )

    mx, nodes_new_flat = pl.pallas_call(
        _gcm_kernel,
        grid_spec=grid_spec,
        out_shape=[
            jax.ShapeDtypeStruct((Bsz, F), jnp.float32),
            jax.ShapeDtypeStruct((Bsz * N, F), jnp.float32),
        ],
    )(nn, x, W, nodes_flat, adj, weights)

    return (mx, nodes_new_flat.reshape(Bsz, N, F), adj, weights,
            num_nodes + 1)
